# baseline (device time: 46241 ns/iter reference)
import jax
import jax.numpy as jnp
from jax import lax
from jax.experimental import pallas as pl
from jax.experimental.pallas import tpu as pltpu

N_DEV = 4
B = 4
SQ = 256
D = 1024
H = 8
DH = 128
T = B * SQ
HALF = T // 2
QTR = T // 4
EGT = T // 8
SCALE = 0.08838834764831843

(PH1A, PH1B, PH2A, PH2B, PH3A, PH3B,
 PH4A1, PH4A2, PH4B1, PH4B2) = range(10)


def kernel(x, Wq, Wo, Wk, Wv):
    def body(x_ref, wq_ref, wo_ref, wk_ref, wv_ref, out_ref,
             attn_ref, acc_ref, p16_ref, rx1_ref, tx2_ref, rx2_ref,
             fin_ref, send_sems, recv_sems):
        p = lax.axis_index("i")
        xc = p // 2
        yc = (p + xc) % 2
        py = p + 1 - 2 * (p % 2)
        px = 3 - p

        a1 = yc
        a2 = xc
        b1 = xc
        b2 = yc

        barrier_sem = pltpu.get_barrier_semaphore()
        for nbr in [py, px]:
            pl.semaphore_signal(
                barrier_sem, inc=1,
                device_id=(nbr,), device_id_type=pl.DeviceIdType.MESH,
            )
        pl.semaphore_wait(barrier_sem, 2)

        wq_bf = wq_ref[...].astype(jnp.bfloat16)
        wk_bf = wk_ref[...].astype(jnp.bfloat16)
        wv_bf = wv_ref[...].astype(jnp.bfloat16)
        wo_bf = wo_ref[...].astype(jnp.bfloat16)

        def compute_batch(batch):
            roff = batch * SQ
            xb = x_ref[pl.ds(batch, 1)][...].reshape(SQ, D).astype(jnp.bfloat16)
            qb = jnp.dot(xb, wq_bf,
                         preferred_element_type=jnp.float32).astype(jnp.bfloat16)
            kb = jnp.dot(xb, wk_bf,
                         preferred_element_type=jnp.float32).astype(jnp.bfloat16)
            vb = jnp.dot(xb, wv_bf,
                         preferred_element_type=jnp.float32).astype(jnp.bfloat16)
            for h in range(H):
                cols = slice(h * DH, (h + 1) * DH)
                s = lax.dot_general(
                    qb[:, cols], kb[:, cols],
                    dimension_numbers=(((1,), (1,)), ((), ())),
                    preferred_element_type=jnp.float32,
                ) * SCALE
                pexp = jnp.exp(s)
                l = jnp.sum(pexp, axis=-1, keepdims=True)
                obh = jnp.dot(pexp.astype(jnp.bfloat16), vb[:, cols],
                              preferred_element_type=jnp.float32)
                attn_ref[:, cols] = (obh / l).astype(jnp.bfloat16)
            partial_b = jnp.dot(attn_ref[...], wo_bf,
                                preferred_element_type=jnp.float32)
            acc_ref[pl.ds(roff, SQ), :] = partial_b
            p16_ref[pl.ds(roff, SQ), :] = partial_b.astype(jnp.bfloat16)

        def exch(src, dst, sem_idx, target):
            return pltpu.make_async_remote_copy(
                src_ref=src, dst_ref=dst,
                send_sem=send_sems.at[sem_idx],
                recv_sem=recv_sems.at[sem_idx],
                device_id=(target,), device_id_type=pl.DeviceIdType.MESH,
            )

        compute_batch(1 - a1)
        compute_batch(3 - b1)
        r1a = exch(p16_ref.at[pl.ds((1 - a1) * QTR, QTR), :],
                   rx1_ref.at[0:QTR, :], PH1A, py)
        r1b = exch(p16_ref.at[pl.ds(HALF + (1 - b1) * QTR, QTR), :],
                   rx1_ref.at[QTR:2 * QTR, :], PH1B, px)
        r1a.start()
        r1b.start()

        compute_batch(a1)
        compute_batch(2 + b1)

        ka = pl.ds(a1 * QTR, QTR)
        kb_ = pl.ds(HALF + b1 * QTR, QTR)
        oa = pl.ds(a1 * QTR + a2 * EGT, EGT)
        ob = pl.ds(HALF + b1 * QTR + b2 * EGT, EGT)
        sa2 = pl.ds(a1 * QTR + (1 - a2) * EGT, EGT)
        sb2 = pl.ds(HALF + b1 * QTR + (1 - b2) * EGT, EGT)

        r1a.wait()
        acc_ref[ka, :] = acc_ref[ka, :] + rx1_ref[0:QTR, :].astype(jnp.float32)
        tx2_ref[0:EGT, :] = acc_ref[sa2, :].astype(jnp.bfloat16)
        r2a = exch(tx2_ref.at[0:EGT, :], rx2_ref.at[0:EGT, :], PH2A, px)
        r2a.start()
        r1b.wait()
        acc_ref[kb_, :] = acc_ref[kb_, :] + rx1_ref[QTR:2 * QTR, :].astype(jnp.float32)
        tx2_ref[EGT:2 * EGT, :] = acc_ref[sb2, :].astype(jnp.bfloat16)
        r2b = exch(tx2_ref.at[EGT:2 * EGT, :], rx2_ref.at[EGT:2 * EGT, :],
                   PH2B, py)
        r2b.start()

        r2a.wait()
        acc_ref[oa, :] = acc_ref[oa, :] + rx2_ref[0:EGT, :].astype(jnp.float32)
        fin_ref[oa, :] = acc_ref[oa, :].astype(jnp.bfloat16)
        r3a = exch(fin_ref.at[oa, :], fin_ref.at[oa, :], PH3A, px)
        r4a1 = exch(fin_ref.at[oa, :], fin_ref.at[oa, :], PH4A1, py)
        r3a.start()
        r4a1.start()
        r2b.wait()
        acc_ref[ob, :] = acc_ref[ob, :] + rx2_ref[EGT:2 * EGT, :].astype(jnp.float32)
        fin_ref[ob, :] = acc_ref[ob, :].astype(jnp.bfloat16)
        r3b = exch(fin_ref.at[ob, :], fin_ref.at[ob, :], PH3B, py)
        r4b1 = exch(fin_ref.at[ob, :], fin_ref.at[ob, :], PH4B1, px)
        r3b.start()
        r4b1.start()

        r3a.wait()
        r4a2 = exch(fin_ref.at[sa2, :], fin_ref.at[sa2, :], PH4A2, py)
        r4a2.start()
        r3b.wait()
        r4b2 = exch(fin_ref.at[sb2, :], fin_ref.at[sb2, :], PH4B2, px)
        r4b2.start()
        r4a1.wait()
        r4b1.wait()
        r4a2.wait()
        r4b2.wait()

        for b in range(B):
            out_ref[b] = fin_ref[b * SQ:(b + 1) * SQ, :].astype(jnp.float32)

    return pl.pallas_call(
        body,
        out_shape=jax.ShapeDtypeStruct((B, SQ, D), jnp.float32),
        in_specs=[pl.BlockSpec(memory_space=pltpu.VMEM)] * 5,
        out_specs=pl.BlockSpec(memory_space=pltpu.VMEM),
        scratch_shapes=[
            pltpu.VMEM((SQ, D), jnp.bfloat16),
            pltpu.VMEM((T, D), jnp.float32),
            pltpu.VMEM((T, D), jnp.bfloat16),
            pltpu.VMEM((HALF, D), jnp.bfloat16),
            pltpu.VMEM((QTR, D), jnp.bfloat16),
            pltpu.VMEM((QTR, D), jnp.bfloat16),
            pltpu.VMEM((T, D), jnp.bfloat16),
            pltpu.SemaphoreType.DMA((10,)),
            pltpu.SemaphoreType.DMA((10,)),
        ],
        compiler_params=pltpu.CompilerParams(collective_id=0),
    )(x, Wq, Wo, Wk, Wv)


# device time: 45688 ns/iter; 1.0121x vs baseline; 1.0121x over previous
import jax
import jax.numpy as jnp
from jax import lax
from jax.experimental import pallas as pl
from jax.experimental.pallas import tpu as pltpu

N_DEV = 4
B = 4
SQ = 256
D = 1024
H = 8
DH = 128
T = B * SQ
HALF = T // 2
QTR = T // 4
EGT = T // 8
SCALE = 0.08838834764831843

(PH1A, PH1B, PH2A, PH2B, PH3A, PH3B,
 PH4A1, PH4A2, PH4B1, PH4B2) = range(10)


def kernel(x, Wq, Wo, Wk, Wv):
    def body(x_ref, wq_ref, wo_ref, wk_ref, wv_ref, out_ref,
             attn_ref, acc_ref, p16_ref, rx1_ref, tx2_ref, rx2_ref,
             fin_ref, send_sems, recv_sems):
        p = lax.axis_index("i")
        xc = p // 2
        yc = (p + xc) % 2
        py = p + 1 - 2 * (p % 2)
        px = 3 - p

        a1 = yc
        a2 = xc
        b1 = xc
        b2 = yc

        barrier_sem = pltpu.get_barrier_semaphore()
        for nbr in [py, px]:
            pl.semaphore_signal(
                barrier_sem, inc=1,
                device_id=(nbr,), device_id_type=pl.DeviceIdType.MESH,
            )
        pl.semaphore_wait(barrier_sem, 2)

        wq_bf = wq_ref[...]
        wk_bf = wk_ref[...]
        wv_bf = wv_ref[...]
        wo_bf = wo_ref[...]

        def compute_batch(batch):
            roff = batch * SQ
            xb = x_ref[pl.ds(batch, 1)][...].reshape(SQ, D)
            qb = jnp.dot(xb, wq_bf,
                         preferred_element_type=jnp.float32).astype(jnp.bfloat16)
            kb = jnp.dot(xb, wk_bf,
                         preferred_element_type=jnp.float32).astype(jnp.bfloat16)
            vb = jnp.dot(xb, wv_bf,
                         preferred_element_type=jnp.float32).astype(jnp.bfloat16)
            for h in range(H):
                cols = slice(h * DH, (h + 1) * DH)
                s = lax.dot_general(
                    qb[:, cols], kb[:, cols],
                    dimension_numbers=(((1,), (1,)), ((), ())),
                    preferred_element_type=jnp.float32,
                ) * SCALE
                pexp = jnp.exp(s)
                l = jnp.sum(pexp, axis=-1, keepdims=True)
                obh = jnp.dot(pexp.astype(jnp.bfloat16), vb[:, cols],
                              preferred_element_type=jnp.float32)
                attn_ref[:, cols] = (obh / l).astype(jnp.bfloat16)
            partial_b = jnp.dot(attn_ref[...], wo_bf,
                                preferred_element_type=jnp.float32)
            acc_ref[pl.ds(roff, SQ), :] = partial_b
            p16_ref[pl.ds(roff, SQ), :] = partial_b.astype(jnp.bfloat16)

        def exch(src, dst, sem_idx, target):
            return pltpu.make_async_remote_copy(
                src_ref=src, dst_ref=dst,
                send_sem=send_sems.at[sem_idx],
                recv_sem=recv_sems.at[sem_idx],
                device_id=(target,), device_id_type=pl.DeviceIdType.MESH,
            )

        compute_batch(1 - a1)
        compute_batch(3 - b1)
        r1a = exch(p16_ref.at[pl.ds((1 - a1) * QTR, QTR), :],
                   rx1_ref.at[0:QTR, :], PH1A, py)
        r1b = exch(p16_ref.at[pl.ds(HALF + (1 - b1) * QTR, QTR), :],
                   rx1_ref.at[QTR:2 * QTR, :], PH1B, px)
        r1a.start()
        r1b.start()

        compute_batch(a1)
        compute_batch(2 + b1)

        ka = pl.ds(a1 * QTR, QTR)
        kb_ = pl.ds(HALF + b1 * QTR, QTR)
        oa = pl.ds(a1 * QTR + a2 * EGT, EGT)
        ob = pl.ds(HALF + b1 * QTR + b2 * EGT, EGT)
        sa2 = pl.ds(a1 * QTR + (1 - a2) * EGT, EGT)
        sb2 = pl.ds(HALF + b1 * QTR + (1 - b2) * EGT, EGT)

        r1a.wait()
        acc_ref[ka, :] = acc_ref[ka, :] + rx1_ref[0:QTR, :].astype(jnp.float32)
        tx2_ref[0:EGT, :] = acc_ref[sa2, :].astype(jnp.bfloat16)
        r2a = exch(tx2_ref.at[0:EGT, :], rx2_ref.at[0:EGT, :], PH2A, px)
        r2a.start()
        r1b.wait()
        acc_ref[kb_, :] = acc_ref[kb_, :] + rx1_ref[QTR:2 * QTR, :].astype(jnp.float32)
        tx2_ref[EGT:2 * EGT, :] = acc_ref[sb2, :].astype(jnp.bfloat16)
        r2b = exch(tx2_ref.at[EGT:2 * EGT, :], rx2_ref.at[EGT:2 * EGT, :],
                   PH2B, py)
        r2b.start()

        r2a.wait()
        acc_ref[oa, :] = acc_ref[oa, :] + rx2_ref[0:EGT, :].astype(jnp.float32)
        fin_ref[oa, :] = acc_ref[oa, :].astype(jnp.bfloat16)
        r3a = exch(fin_ref.at[oa, :], fin_ref.at[oa, :], PH3A, px)
        r4a1 = exch(fin_ref.at[oa, :], fin_ref.at[oa, :], PH4A1, py)
        r3a.start()
        r4a1.start()
        r2b.wait()
        acc_ref[ob, :] = acc_ref[ob, :] + rx2_ref[EGT:2 * EGT, :].astype(jnp.float32)
        fin_ref[ob, :] = acc_ref[ob, :].astype(jnp.bfloat16)
        r3b = exch(fin_ref.at[ob, :], fin_ref.at[ob, :], PH3B, py)
        r4b1 = exch(fin_ref.at[ob, :], fin_ref.at[ob, :], PH4B1, px)
        r3b.start()
        r4b1.start()

        r3a.wait()
        r4a2 = exch(fin_ref.at[sa2, :], fin_ref.at[sa2, :], PH4A2, py)
        r4a2.start()
        r3b.wait()
        r4b2 = exch(fin_ref.at[sb2, :], fin_ref.at[sb2, :], PH4B2, px)
        r4b2.start()
        r4a1.wait()
        r4b1.wait()
        r4a2.wait()
        r4b2.wait()

        for b in range(B):
            out_ref[b] = fin_ref[b * SQ:(b + 1) * SQ, :].astype(jnp.float32)

    return pl.pallas_call(
        body,
        out_shape=jax.ShapeDtypeStruct((B, SQ, D), jnp.float32),
        in_specs=[pl.BlockSpec(memory_space=pltpu.VMEM)] * 5,
        out_specs=pl.BlockSpec(memory_space=pltpu.VMEM),
        scratch_shapes=[
            pltpu.VMEM((SQ, D), jnp.bfloat16),
            pltpu.VMEM((T, D), jnp.float32),
            pltpu.VMEM((T, D), jnp.bfloat16),
            pltpu.VMEM((HALF, D), jnp.bfloat16),
            pltpu.VMEM((QTR, D), jnp.bfloat16),
            pltpu.VMEM((QTR, D), jnp.bfloat16),
            pltpu.VMEM((T, D), jnp.bfloat16),
            pltpu.SemaphoreType.DMA((10,)),
            pltpu.SemaphoreType.DMA((10,)),
        ],
        compiler_params=pltpu.CompilerParams(collective_id=0),
    )(x.astype(jnp.bfloat16), Wq.astype(jnp.bfloat16),
      Wo.astype(jnp.bfloat16), Wk.astype(jnp.bfloat16),
      Wv.astype(jnp.bfloat16))
